# baseline (speedup 1.0000x reference)
import functools

import jax
import jax.numpy as jnp
from jax import lax
from jax.experimental import pallas as pl
from jax.experimental.pallas import tpu as pltpu
from jax.experimental.pallas import tpu_sc as plsc


def _sc_spmm(table, pk, ew, n_nodes, n_ch, B, D, AH, split_ch=False):
    info = plsc.get_sparse_core_info()
    NC, NS = info.num_cores, info.num_subcores
    NW = NC * NS
    NB = pk.shape[1]
    assert NB % D == 0 and 1 <= AH <= D - 1
    RPS = n_nodes // NS
    CZ = n_ch // 16

    mesh = plsc.VectorSubcoreMesh(core_axis_name="c", subcore_axis_name="s")

    @functools.partial(
        pl.kernel,
        mesh=mesh,
        compiler_params=pltpu.CompilerParams(use_tc_tiling_on_sc=False),
        out_type=jax.ShapeDtypeStruct((NC * n_nodes, n_ch), jnp.float32),
        scratch_types=[
            pltpu.VMEM((NB, B), jnp.int32),
            pltpu.VMEM((D, B, n_ch), jnp.float32),
            pltpu.VMEM((D, B), jnp.int32),
            pltpu.VMEM((D, B), jnp.int32),
            pltpu.VMEM((D, B), jnp.float32),
            pltpu.VMEM_SHARED((n_nodes, n_ch), jnp.float32),
            pltpu.SemaphoreType.DMA((D,)),
            pltpu.SemaphoreType.DMA((D,)),
        ],
    )
    def spmm_kernel(tab_hbm, pk_hbm, ew_hbm, zeros_hbm, out_hbm,
                    pks, rows, sidx, didx, wv, acc, gsem, ssem):
        c = lax.axis_index("c")
        s = lax.axis_index("s")
        wid = s if split_ch else s * NC + c

        pltpu.sync_copy(pk_hbm.at[wid], pks)

        def issue(j, k):
            for g in range(B // 16):
                v = pks[j, pl.ds(16 * g, 16)]
                sv = v & 0xFFFF
                if split_ch:
                    sv = sv + c * n_nodes
                sidx[k, pl.ds(16 * g, 16)] = sv
                didx[k, pl.ds(16 * g, 16)] = lax.shift_right_logical(v, 16)
            pltpu.async_copy(ew_hbm.at[wid, j], wv.at[k], gsem.at[k])
            pltpu.async_copy(tab_hbm.at[sidx.at[k]], rows.at[k], gsem.at[k])

        def wait_gather(k):
            pltpu.make_async_copy(tab_hbm.at[pl.ds(0, B)], rows.at[k],
                                  gsem.at[k]).wait()
            pltpu.make_async_copy(ew_hbm.at[0, 0], wv.at[k],
                                  gsem.at[k]).wait()

        def wait_scatter(k):
            pltpu.make_async_copy(rows.at[k], acc.at[pl.ds(0, B)],
                                  ssem.at[k]).wait()

        def scale(k, buf):
            @plsc.parallel_loop(0, B // 16, unroll=2)
            def _scale(g):
                wch = wv[k, pl.ds(16 * g, 16)]
                for r in range(16):
                    w = jnp.full((16,), wch[r], jnp.float32)
                    row = 16 * g + r
                    for j in range(CZ):
                        buf[row, pl.ds(16 * j, 16)] = (
                            buf[row, pl.ds(16 * j, 16)] * w)

        for j in range(AH):
            issue(j, j)

        pltpu.sync_copy(zeros_hbm, acc.at[pl.ds(s * RPS, RPS)])
        plsc.subcore_barrier()

        @pl.loop(0, NB // D)
        def _edges(t):
            i0 = t * D
            for k in range(D):
                i = i0 + k
                wait_gather(k)
                scale(k, rows.at[k])
                pltpu.async_copy(rows.at[k], acc.at[didx.at[k]],
                                 ssem.at[k], add=True)
                kf = (k + AH) % D

                @pl.when(i + AH < NB)
                def _():
                    @pl.when(i + AH >= D)
                    def _():
                        wait_scatter(kf)
                    issue(i + AH, kf)

        for k in range(D):
            wait_scatter(k)

        plsc.subcore_barrier()

        pltpu.sync_copy(acc.at[pl.ds(s * RPS, RPS)],
                        out_hbm.at[pl.ds(c * n_nodes + s * RPS, RPS)])

    out = spmm_kernel(table, pk, ew, jnp.zeros((RPS, n_ch), jnp.float32))
    return out.reshape(NC, n_nodes, n_ch)


def _pack_edges(src, dst, ew, B, D, workers):
    e = src.shape[0]
    nb = -(-e // (workers * B))
    nb = ((nb + D - 1) // D) * D
    pad = workers * nb * B - e
    pk = jnp.bitwise_or(src, jnp.left_shift(dst, 16))
    pk = jnp.concatenate([pk, jnp.zeros((pad,), jnp.int32)])
    ewp = jnp.concatenate([ew, jnp.zeros((pad,), jnp.float32)])
    return pk.reshape(workers, nb, B), ewp.reshape(workers, nb, B)


def _mm_body(x_ref, w_ref, o_ref):
    o_ref[...] = jnp.dot(x_ref[...], w_ref[...],
                         preferred_element_type=jnp.float32)


def _tc_matmul(x, w, blk):
    n, kdim = x.shape
    m = w.shape[1]
    grid = n // blk
    return pl.pallas_call(
        _mm_body,
        grid=(grid,),
        in_specs=[
            pl.BlockSpec((blk, kdim), lambda i: (i, 0)),
            pl.BlockSpec((kdim, m), lambda i: (0, 0)),
        ],
        out_specs=pl.BlockSpec((blk, m), lambda i: (i, 0)),
        out_shape=jax.ShapeDtypeStruct((n, m), jnp.float32),
    )(x, w)


def _merge_body(p_ref, b_ref, w_ref, o_ref):
    h = jnp.maximum(p_ref[0] + p_ref[1] + b_ref[...], 0.0)
    o_ref[...] = jnp.dot(h, w_ref[...], preferred_element_type=jnp.float32)


def _tc_merge_relu_mm(partials, b1, w2b, blk):
    _, n, kdim = partials.shape
    m = w2b.shape[1]
    grid = n // blk
    return pl.pallas_call(
        _merge_body,
        grid=(grid,),
        in_specs=[
            pl.BlockSpec((2, blk, kdim), lambda i: (0, i, 0)),
            pl.BlockSpec((1, kdim), lambda i: (0, 0)),
            pl.BlockSpec((kdim, m), lambda i: (0, 0)),
        ],
        out_specs=pl.BlockSpec((blk, m), lambda i: (i, 0)),
        out_shape=jax.ShapeDtypeStruct((n, m), jnp.float32),
    )(partials, b1, w2b)


def _lsm_body(p_ref, o_ref):
    s = p_ref[0] + p_ref[1]
    m = jnp.max(s)
    e = jnp.exp(s - m)
    t = jnp.sum(e) * (1.0 / 16.0)
    o_ref[...] = s - (m + jnp.log(t))


def _tc_log_softmax(partials):
    _, n, m = partials.shape
    return pl.pallas_call(
        _lsm_body,
        out_shape=jax.ShapeDtypeStruct((n, m), jnp.float32),
    )(partials)


def kernel(features, edge_index, edge_weight, W1, b1, W2, b2):
    n = features.shape[0]
    src = edge_index[0].astype(jnp.int32)
    dst = edge_index[1].astype(jnp.int32)
    ew = edge_weight.astype(jnp.float32)

    support = _tc_matmul(features, W1, blk=1000)

    pk1, ew1 = _pack_edges(src, dst, ew, B=128, D=2, workers=32)
    p1 = _sc_spmm(support, pk1, ew1, n, support.shape[1], B=128, D=2, AH=1)

    w2b = jnp.tile(W2, (1, 16))
    y16 = _tc_merge_relu_mm(p1, b1.reshape(1, -1), w2b, blk=1000)

    pk2, ew2 = _pack_edges(src, dst, ew, B=128, D=8, workers=32)
    p2 = _sc_spmm(y16, pk2, ew2, n, 16, B=128, D=8, AH=6)

    out16 = _tc_log_softmax(p2)
    return out16[:, :1]


# speedup vs baseline: 2.0298x; 2.0298x over previous
import functools

import jax
import jax.numpy as jnp
from jax import lax
from jax.experimental import pallas as pl
from jax.experimental.pallas import tpu as pltpu
from jax.experimental.pallas import tpu_sc as plsc


def _sc_spmm(table, pk, ew, n_nodes, n_ch, B, D, AH, split_ch=False):
    info = plsc.get_sparse_core_info()
    NC, NS = info.num_cores, info.num_subcores
    NW = NC * NS
    NB = pk.shape[1]
    assert NB % D == 0 and 1 <= AH <= D - 1
    RPS = n_nodes // NS
    CZ = n_ch // 16

    mesh = plsc.VectorSubcoreMesh(core_axis_name="c", subcore_axis_name="s")

    @functools.partial(
        pl.kernel,
        mesh=mesh,
        compiler_params=pltpu.CompilerParams(use_tc_tiling_on_sc=False),
        out_type=jax.ShapeDtypeStruct((NC * n_nodes, n_ch), jnp.float32),
        scratch_types=[
            pltpu.VMEM((NB, B), jnp.int32),
            pltpu.VMEM((D, B, n_ch), jnp.float32),
            pltpu.VMEM((D, B), jnp.int32),
            pltpu.VMEM((D, B), jnp.int32),
            pltpu.VMEM((D, B), jnp.float32),
            pltpu.VMEM_SHARED((n_nodes, n_ch), jnp.float32),
            pltpu.SemaphoreType.DMA((D,)),
            pltpu.SemaphoreType.DMA((D,)),
        ],
    )
    def spmm_kernel(tab_hbm, pk_hbm, ew_hbm, zeros_hbm, out_hbm,
                    pks, rows, sidx, didx, wv, acc, gsem, ssem):
        c = lax.axis_index("c")
        s = lax.axis_index("s")
        wid = s if split_ch else s * NC + c

        pltpu.sync_copy(pk_hbm.at[wid], pks)

        def issue(j, k):
            for g in range(B // 16):
                v = pks[j, pl.ds(16 * g, 16)]
                sv = v & 0xFFFF
                if split_ch:
                    sv = sv + c * n_nodes
                sidx[k, pl.ds(16 * g, 16)] = sv
                didx[k, pl.ds(16 * g, 16)] = lax.shift_right_logical(v, 16)
            pltpu.async_copy(ew_hbm.at[wid, j], wv.at[k], gsem.at[k])
            pltpu.async_copy(tab_hbm.at[sidx.at[k]], rows.at[k], gsem.at[k])

        def wait_gather(k):
            pltpu.make_async_copy(tab_hbm.at[pl.ds(0, B)], rows.at[k],
                                  gsem.at[k]).wait()
            pltpu.make_async_copy(ew_hbm.at[0, 0], wv.at[k],
                                  gsem.at[k]).wait()

        def wait_scatter(k):
            pltpu.make_async_copy(rows.at[k], acc.at[pl.ds(0, B)],
                                  ssem.at[k]).wait()

        def scale(k, buf):
            @plsc.parallel_loop(0, B // 16, unroll=2)
            def _scale(g):
                wch = wv[k, pl.ds(16 * g, 16)]
                for r in range(16):
                    w = jnp.full((16,), wch[r], jnp.float32)
                    row = 16 * g + r
                    for j in range(CZ):
                        buf[row, pl.ds(16 * j, 16)] = (
                            buf[row, pl.ds(16 * j, 16)] * w)

        for j in range(AH):
            issue(j, j)

        pltpu.sync_copy(zeros_hbm, acc.at[pl.ds(s * RPS, RPS)])
        plsc.subcore_barrier()

        @pl.loop(0, NB // D)
        def _edges(t):
            i0 = t * D
            for k in range(D):
                i = i0 + k
                wait_gather(k)
                scale(k, rows.at[k])
                pltpu.async_copy(rows.at[k], acc.at[didx.at[k]],
                                 ssem.at[k], add=True)
                kf = (k + AH) % D

                @pl.when(i + AH < NB)
                def _():
                    @pl.when(i + AH >= D)
                    def _():
                        wait_scatter(kf)
                    issue(i + AH, kf)

        for k in range(D):
            wait_scatter(k)

        plsc.subcore_barrier()

        pltpu.sync_copy(acc.at[pl.ds(s * RPS, RPS)],
                        out_hbm.at[pl.ds(c * n_nodes + s * RPS, RPS)])

    out = spmm_kernel(table, pk, ew, jnp.zeros((RPS, n_ch), jnp.float32))
    return out.reshape(NC, n_nodes, n_ch)


def _pack_edges(src, dst, ew, B, D, workers):
    e = src.shape[0]
    nb = -(-e // (workers * B))
    nb = ((nb + D - 1) // D) * D
    pad = workers * nb * B - e
    pk = jnp.bitwise_or(src, jnp.left_shift(dst, 16))
    pk = jnp.concatenate([pk, jnp.zeros((pad,), jnp.int32)])
    ewp = jnp.concatenate([ew, jnp.zeros((pad,), jnp.float32)])
    return pk.reshape(workers, nb, B), ewp.reshape(workers, nb, B)


def _mm_body(x_ref, w_ref, o_ref):
    o_ref[...] = jnp.dot(x_ref[...], w_ref[...],
                         preferred_element_type=jnp.float32)


def _tc_matmul(x, w, blk):
    n, kdim = x.shape
    m = w.shape[1]
    grid = n // blk
    return pl.pallas_call(
        _mm_body,
        grid=(grid,),
        in_specs=[
            pl.BlockSpec((blk, kdim), lambda i: (i, 0)),
            pl.BlockSpec((kdim, m), lambda i: (0, 0)),
        ],
        out_specs=pl.BlockSpec((blk, m), lambda i: (i, 0)),
        out_shape=jax.ShapeDtypeStruct((n, m), jnp.float32),
    )(x, w)


def _merge_body(p_ref, b_ref, w_ref, o_ref):
    h = jnp.maximum(p_ref[0] + p_ref[1] + b_ref[...], 0.0)
    o_ref[...] = jnp.dot(h, w_ref[...], preferred_element_type=jnp.float32)


def _tc_merge_relu_mm(partials, b1, w2b, blk):
    _, n, kdim = partials.shape
    m = w2b.shape[1]
    grid = n // blk
    return pl.pallas_call(
        _merge_body,
        grid=(grid,),
        in_specs=[
            pl.BlockSpec((2, blk, kdim), lambda i: (0, i, 0)),
            pl.BlockSpec((1, kdim), lambda i: (0, 0)),
            pl.BlockSpec((kdim, m), lambda i: (0, 0)),
        ],
        out_specs=pl.BlockSpec((blk, m), lambda i: (i, 0)),
        out_shape=jax.ShapeDtypeStruct((n, m), jnp.float32),
    )(partials, b1, w2b)


def _lsm_body(p_ref, o_ref):
    s = p_ref[0] + p_ref[1]
    m = jnp.max(s)
    e = jnp.exp(s - m)
    t = jnp.sum(e) * (1.0 / 16.0)
    o_ref[...] = s - (m + jnp.log(t))


def _tc_log_softmax(partials):
    _, n, m = partials.shape
    return pl.pallas_call(
        _lsm_body,
        out_shape=jax.ShapeDtypeStruct((n, m), jnp.float32),
    )(partials)


def kernel(features, edge_index, edge_weight, W1, b1, W2, b2):
    n = features.shape[0]
    src = edge_index[0].astype(jnp.int32)
    dst = edge_index[1].astype(jnp.int32)
    ew = edge_weight.astype(jnp.float32)

    support = _tc_matmul(features, W1, blk=1000)

    pk1, ew1 = _pack_edges(src, dst, ew, B=80, D=3, workers=32)
    p1 = _sc_spmm(support, pk1, ew1, n, support.shape[1], B=80, D=3, AH=2)

    w2b = jnp.tile(W2, (1, 16))
    y16 = _tc_merge_relu_mm(p1, b1.reshape(1, -1), w2b, blk=1000)

    pk2, ew2 = _pack_edges(src, dst, ew, B=128, D=8, workers=32)
    p2 = _sc_spmm(y16, pk2, ew2, n, 16, B=128, D=8, AH=6)

    out16 = _tc_log_softmax(p2)
    return out16[:, :1]
